# baseline (speedup 1.0000x reference)
import functools
import math

import jax
import jax.numpy as jnp
from jax import lax
from jax.experimental import pallas as pl
from jax.experimental.pallas import tpu as pltpu
from jax.experimental.pallas import tpu_sc as plsc

N_NODES = 10000
D = 128
N_EDGES = 320000
INV_BN = 1.0 / math.sqrt(1.0 + 1e-5)

NC = 2
NS = 16
NW = NC * NS

EC = 128
E_PAD = 327680
CHUNKS = E_PAD // (NW * EC)
HALF = CHUNKS // 2
PAD_ROW = 10016
N_PAD = 10240
ROWS_PER_TILE = N_PAD // NS

BPAD = 10240
GC = 80
GCHUNKS = BPAD // (NW * GC)

_sc_mesh = plsc.VectorSubcoreMesh(core_axis_name="c", subcore_axis_name="s")


def _segment_sum_sc(features, tgt1d, src1d):
    @functools.partial(
        pl.kernel,
        out_type=jax.ShapeDtypeStruct((NC, N_PAD, D), jnp.float32),
        mesh=_sc_mesh,
        scratch_types=[
            pltpu.VMEM((EC,), jnp.int32),
            pltpu.VMEM((EC,), jnp.int32),
            pltpu.VMEM((EC,), jnp.int32),
            pltpu.VMEM((EC,), jnp.int32),
            pltpu.VMEM_SHARED((N_PAD, D), jnp.float32),
            pltpu.VMEM((EC, D), jnp.float32),
            pltpu.VMEM((EC, D), jnp.float32),
            pltpu.SemaphoreType.DMA,
            pltpu.SemaphoreType.DMA,
        ],
    )
    def k(feat_hbm, tgt_hbm, src_hbm, out_hbm, tgt0, tgt1, src0, src1,
          acc_sh, rows0, rows1, sem0, sem1):
        c = lax.axis_index("c")
        s = lax.axis_index("s")
        w = c * NS + s

        def zrow(i, carry):
            for j in range(D // 16):
                rows0[i, pl.ds(j * 16, 16)] = jnp.zeros((16,), jnp.float32)
            return carry

        lax.fori_loop(0, EC, zrow, 0)
        r0 = s * ROWS_PER_TILE
        for j in range(ROWS_PER_TILE // EC):
            pltpu.sync_copy(rows0, acc_sh.at[pl.ds(r0 + j * EC, EC)])
        plsc.subcore_barrier()

        ebase = w * CHUNKS * EC

        def g0wait():
            pltpu.make_async_copy(feat_hbm.at[tgt0], rows0, sem0).wait()

        def g1wait():
            pltpu.make_async_copy(feat_hbm.at[tgt1], rows1, sem1).wait()

        pltpu.sync_copy(tgt_hbm.at[pl.ds(ebase, EC)], tgt0)
        pltpu.sync_copy(src_hbm.at[pl.ds(ebase, EC)], src0)
        pltpu.async_copy(feat_hbm.at[tgt0], rows0, sem0)
        pltpu.sync_copy(tgt_hbm.at[pl.ds(ebase + EC, EC)], tgt1)
        pltpu.sync_copy(src_hbm.at[pl.ds(ebase + EC, EC)], src1)
        pltpu.async_copy(feat_hbm.at[tgt1], rows1, sem1)

        def body2(jj, carry):
            j0 = jj * 2
            g0wait()
            pltpu.sync_copy(rows0, acc_sh.at[src0], add=True)
            pltpu.sync_copy(tgt_hbm.at[pl.ds(ebase + (j0 + 2) * EC, EC)], tgt0)
            pltpu.sync_copy(src_hbm.at[pl.ds(ebase + (j0 + 2) * EC, EC)], src0)
            pltpu.async_copy(feat_hbm.at[tgt0], rows0, sem0)
            g1wait()
            pltpu.sync_copy(rows1, acc_sh.at[src1], add=True)
            pltpu.sync_copy(tgt_hbm.at[pl.ds(ebase + (j0 + 3) * EC, EC)], tgt1)
            pltpu.sync_copy(src_hbm.at[pl.ds(ebase + (j0 + 3) * EC, EC)], src1)
            pltpu.async_copy(feat_hbm.at[tgt1], rows1, sem1)
            return carry

        lax.fori_loop(0, CHUNKS // 2 - 1, body2, 0)
        g0wait()
        pltpu.sync_copy(rows0, acc_sh.at[src0], add=True)
        g1wait()
        pltpu.sync_copy(rows1, acc_sh.at[src1], add=True)
        plsc.subcore_barrier()

        pltpu.sync_copy(acc_sh.at[pl.ds(r0, ROWS_PER_TILE)],
                        out_hbm.at[c, pl.ds(r0, ROWS_PER_TILE)])

    return k(features, tgt1d, src1d)


def _dense_tc(features, partials, W1, W2, b, gamma, beta):
    R = 1000

    def body(f_ref, p_ref, w1_ref, w2_ref, b_ref, g_ref, bt_ref, o_ref):
        x = f_ref[...]
        a = p_ref[0] + p_ref[1]
        dn = (((1,), (1,)), ((), ()))
        y = lax.dot_general(x, w1_ref[...], dn,
                            preferred_element_type=jnp.float32)
        y = y + lax.dot_general(a, w2_ref[...], dn,
                                preferred_element_type=jnp.float32)
        y = y + b_ref[...]
        y = jnp.maximum(y, 0.0)
        y = y * (g_ref[...] * INV_BN) + bt_ref[...]
        n = jnp.sqrt(jnp.sum(y * y, axis=1, keepdims=True))
        o_ref[...] = y / (n + 1e-6)

    return pl.pallas_call(
        body,
        grid=(N_NODES // R,),
        in_specs=[
            pl.BlockSpec((R, D), lambda i: (i, 0)),
            pl.BlockSpec((NC, R, D), lambda i: (0, i, 0)),
            pl.BlockSpec((D, D), lambda i: (0, 0)),
            pl.BlockSpec((D, D), lambda i: (0, 0)),
            pl.BlockSpec((1, D), lambda i: (0, 0)),
            pl.BlockSpec((1, D), lambda i: (0, 0)),
            pl.BlockSpec((1, D), lambda i: (0, 0)),
        ],
        out_specs=pl.BlockSpec((R, D), lambda i: (i, 0)),
        out_shape=jax.ShapeDtypeStruct((N_NODES, D), jnp.float32),
    )(features, partials, W1, W2, b, gamma, beta)


def _gather_sc(u, idx2d):
    @functools.partial(
        pl.kernel,
        out_type=jax.ShapeDtypeStruct((BPAD, D), jnp.float32),
        mesh=_sc_mesh,
        scratch_types=[
            pltpu.VMEM((GCHUNKS, GC), jnp.int32),
            pltpu.VMEM((GC, D), jnp.float32),
            pltpu.VMEM((GC, D), jnp.float32),
            pltpu.SemaphoreType.DMA,
            pltpu.SemaphoreType.DMA,
        ],
    )
    def k(u_hbm, idx_hbm, out_hbm, idx_v, rows0, rows1, sem0, sem1):
        c = lax.axis_index("c")
        s = lax.axis_index("s")
        w = s * NC + c
        base = w * GCHUNKS * GC
        pltpu.sync_copy(idx_hbm.at[pl.ds(w * GCHUNKS, GCHUNKS)], idx_v)
        pltpu.async_copy(u_hbm.at[idx_v.at[0]], rows0, sem0)
        pltpu.async_copy(u_hbm.at[idx_v.at[1]], rows1, sem1)
        for j in range(GCHUNKS):
            rows = rows0 if j % 2 == 0 else rows1
            sem = sem0 if j % 2 == 0 else sem1
            pltpu.make_async_copy(u_hbm.at[idx_v.at[0]], rows, sem).wait()
            pltpu.sync_copy(rows, out_hbm.at[pl.ds(base + j * GC, GC)])
            if j + 2 < GCHUNKS:
                pltpu.async_copy(u_hbm.at[idx_v.at[j + 2]], rows, sem)

    return k(u, idx2d)


def kernel(features, batch, edge_index, W, b, gamma, beta):
    src = edge_index[0].astype(jnp.int32)
    tgt = edge_index[1].astype(jnp.int32)
    bidx = batch.astype(jnp.int32)

    epad = E_PAD - N_EDGES
    src1d = jnp.concatenate([src, jnp.full((epad,), PAD_ROW, jnp.int32)])
    tgt1d = jnp.concatenate([tgt, jnp.zeros((epad,), jnp.int32)])

    partials = _segment_sum_sc(features, tgt1d, src1d)
    W1 = W[:, :D]
    W2 = W[:, D:]
    u = _dense_tc(features, partials, W1, W2, b.reshape(1, D),
                  gamma.reshape(1, D), beta.reshape(1, D))
    bpad = jnp.concatenate(
        [bidx, jnp.zeros((BPAD - N_NODES,), jnp.int32)]).reshape(-1, GC)
    outp = _gather_sc(u, bpad)
    return outp[:N_NODES]


# speedup vs baseline: 1.0015x; 1.0015x over previous
import functools
import math

import jax
import jax.numpy as jnp
from jax import lax
from jax.experimental import pallas as pl
from jax.experimental.pallas import tpu as pltpu
from jax.experimental.pallas import tpu_sc as plsc

N_NODES = 10000
D = 128
N_EDGES = 320000
INV_BN = 1.0 / math.sqrt(1.0 + 1e-5)

NC = 2
NS = 16
NW = NC * NS

EC = 128
E_PAD = 327680
CHUNKS = E_PAD // (NW * EC)
HALF = CHUNKS // 2
PAD_ROW = 10016
N_PAD = 10240
ROWS_PER_TILE = N_PAD // NS

BPAD = 10240
GC = 80
GCHUNKS = BPAD // (NW * GC)

_sc_mesh = plsc.VectorSubcoreMesh(core_axis_name="c", subcore_axis_name="s")


def _segment_sum_sc(features, tgt1d, src1d):
    @functools.partial(
        pl.kernel,
        out_type=jax.ShapeDtypeStruct((NC, N_PAD, D), jnp.float32),
        mesh=_sc_mesh,
        scratch_types=[
            pltpu.VMEM((EC,), jnp.int32),
            pltpu.VMEM((EC,), jnp.int32),
            pltpu.VMEM((EC,), jnp.int32),
            pltpu.VMEM((EC,), jnp.int32),
            pltpu.VMEM_SHARED((N_PAD, D), jnp.float32),
            pltpu.VMEM((EC, D), jnp.float32),
            pltpu.VMEM((EC, D), jnp.float32),
            pltpu.SemaphoreType.DMA,
            pltpu.SemaphoreType.DMA,
        ],
    )
    def k(feat_hbm, tgt_hbm, src_hbm, out_hbm, tgt0, tgt1, src0, src1,
          acc_sh, rows0, rows1, sem0, sem1):
        c = lax.axis_index("c")
        s = lax.axis_index("s")
        w = c * NS + s

        def zrow(i, carry):
            for j in range(D // 16):
                rows0[i, pl.ds(j * 16, 16)] = jnp.zeros((16,), jnp.float32)
            return carry

        lax.fori_loop(0, EC, zrow, 0)
        r0 = s * ROWS_PER_TILE
        for j in range(ROWS_PER_TILE // EC):
            pltpu.sync_copy(rows0, acc_sh.at[pl.ds(r0 + j * EC, EC)])
        plsc.subcore_barrier()

        ebase = w * CHUNKS * EC

        def g0wait():
            pltpu.make_async_copy(feat_hbm.at[tgt0], rows0, sem0).wait()

        def g1wait():
            pltpu.make_async_copy(feat_hbm.at[tgt1], rows1, sem1).wait()

        pltpu.sync_copy(tgt_hbm.at[pl.ds(ebase, EC)], tgt0)
        pltpu.sync_copy(src_hbm.at[pl.ds(ebase, EC)], src0)
        pltpu.async_copy(feat_hbm.at[tgt0], rows0, sem0)
        pltpu.sync_copy(tgt_hbm.at[pl.ds(ebase + EC, EC)], tgt1)
        pltpu.sync_copy(src_hbm.at[pl.ds(ebase + EC, EC)], src1)
        pltpu.async_copy(feat_hbm.at[tgt1], rows1, sem1)

        def body2(jj, carry):
            j0 = jj * 2
            g0wait()
            pltpu.sync_copy(rows0, acc_sh.at[src0], add=True)
            pltpu.sync_copy(tgt_hbm.at[pl.ds(ebase + (j0 + 2) * EC, EC)], tgt0)
            pltpu.sync_copy(src_hbm.at[pl.ds(ebase + (j0 + 2) * EC, EC)], src0)
            pltpu.async_copy(feat_hbm.at[tgt0], rows0, sem0)
            g1wait()
            pltpu.sync_copy(rows1, acc_sh.at[src1], add=True)
            pltpu.sync_copy(tgt_hbm.at[pl.ds(ebase + (j0 + 3) * EC, EC)], tgt1)
            pltpu.sync_copy(src_hbm.at[pl.ds(ebase + (j0 + 3) * EC, EC)], src1)
            pltpu.async_copy(feat_hbm.at[tgt1], rows1, sem1)
            return carry

        lax.fori_loop(0, CHUNKS // 2 - 1, body2, 0)
        g0wait()
        pltpu.sync_copy(rows0, acc_sh.at[src0], add=True)
        g1wait()
        pltpu.sync_copy(rows1, acc_sh.at[src1], add=True)
        plsc.subcore_barrier()

        pltpu.sync_copy(acc_sh.at[pl.ds(r0, ROWS_PER_TILE)],
                        out_hbm.at[c, pl.ds(r0, ROWS_PER_TILE)])

    return k(features, tgt1d, src1d)


def _dense_tc(features, partials, W1, W2, b, gamma, beta):
    R = 1000

    def body(f_ref, p_ref, w1_ref, w2_ref, b_ref, g_ref, bt_ref, o_ref):
        x = f_ref[...]
        a = p_ref[0] + p_ref[1]
        dn = (((1,), (1,)), ((), ()))
        y = lax.dot_general(x, w1_ref[...], dn,
                            preferred_element_type=jnp.float32)
        y = y + lax.dot_general(a, w2_ref[...], dn,
                                preferred_element_type=jnp.float32)
        y = y + b_ref[...]
        y = jnp.maximum(y, 0.0)
        y = y * (g_ref[...] * INV_BN) + bt_ref[...]
        n = jnp.sqrt(jnp.sum(y * y, axis=1, keepdims=True))
        o_ref[...] = y / (n + 1e-6)

    return pl.pallas_call(
        body,
        grid=(N_NODES // R,),
        in_specs=[
            pl.BlockSpec((R, D), lambda i: (i, 0)),
            pl.BlockSpec((NC, R, D), lambda i: (0, i, 0)),
            pl.BlockSpec((D, D), lambda i: (0, 0)),
            pl.BlockSpec((D, D), lambda i: (0, 0)),
            pl.BlockSpec((1, D), lambda i: (0, 0)),
            pl.BlockSpec((1, D), lambda i: (0, 0)),
            pl.BlockSpec((1, D), lambda i: (0, 0)),
        ],
        out_specs=pl.BlockSpec((R, D), lambda i: (i, 0)),
        out_shape=jax.ShapeDtypeStruct((N_NODES, D), jnp.float32),
    )(features, partials, W1, W2, b, gamma, beta)


def _gather_sc(u, idx2d):
    @functools.partial(
        pl.kernel,
        out_type=jax.ShapeDtypeStruct((BPAD, D), jnp.float32),
        mesh=_sc_mesh,
        scratch_types=[
            pltpu.VMEM((GCHUNKS, GC), jnp.int32),
            pltpu.VMEM((GC, D), jnp.float32),
            pltpu.VMEM((GC, D), jnp.float32),
            pltpu.SemaphoreType.DMA,
            pltpu.SemaphoreType.DMA,
        ],
    )
    def k(u_hbm, idx_hbm, out_hbm, idx_v, rows0, rows1, sem0, sem1):
        c = lax.axis_index("c")
        s = lax.axis_index("s")
        w = s * NC + c
        base = w * GCHUNKS * GC
        pltpu.sync_copy(idx_hbm.at[pl.ds(w * GCHUNKS, GCHUNKS)], idx_v)
        pltpu.async_copy(u_hbm.at[idx_v.at[0]], rows0, sem0)
        pltpu.async_copy(u_hbm.at[idx_v.at[1]], rows1, sem1)
        for j in range(GCHUNKS):
            rows = rows0 if j % 2 == 0 else rows1
            sem = sem0 if j % 2 == 0 else sem1
            pltpu.make_async_copy(u_hbm.at[idx_v.at[0]], rows, sem).wait()
            pltpu.sync_copy(rows, out_hbm.at[pl.ds(base + j * GC, GC)])
            if j + 2 < GCHUNKS:
                pltpu.async_copy(u_hbm.at[idx_v.at[j + 2]], rows, sem)

    return k(u, idx2d)


def kernel(features, batch, edge_index, W, b, gamma, beta):
    src = edge_index[0].astype(jnp.int32)
    tgt = edge_index[1].astype(jnp.int32)
    bidx = batch.astype(jnp.int32)

    epad = E_PAD - N_EDGES
    pad_src = N_NODES + (jnp.arange(epad, dtype=jnp.int32) % (N_PAD - N_NODES))
    src1d = jnp.concatenate([src, pad_src])
    tgt1d = jnp.concatenate([tgt, jnp.zeros((epad,), jnp.int32)])

    partials = _segment_sum_sc(features, tgt1d, src1d)
    W1 = W[:, :D]
    W2 = W[:, D:]
    u = _dense_tc(features, partials, W1, W2, b.reshape(1, D),
                  gamma.reshape(1, D), beta.reshape(1, D))
    bpad = jnp.concatenate(
        [bidx, jnp.zeros((BPAD - N_NODES,), jnp.int32)]).reshape(-1, GC)
    outp = _gather_sc(u, bpad)
    return outp[:N_NODES]


# speedup vs baseline: 2.1339x; 2.1306x over previous
import functools
import math

import jax
import jax.numpy as jnp
from jax import lax
from jax.experimental import pallas as pl
from jax.experimental.pallas import tpu as pltpu
from jax.experimental.pallas import tpu_sc as plsc

N_NODES = 10000
D = 128
N_EDGES = 320000
INV_BN = 1.0 / math.sqrt(1.0 + 1e-5)

NC = 2
NS = 16
NW = NC * NS

EC = 128
E_PAD = 327680
CHUNKS = E_PAD // (NW * EC)
HALF = CHUNKS // 2
PAD_ROW = 10016
N_PAD = 10240
ROWS_PER_TILE = N_PAD // NS

BPAD = 10240
GC = 80
GCHUNKS = BPAD // (NW * GC)

_sc_mesh = plsc.VectorSubcoreMesh(core_axis_name="c", subcore_axis_name="s")


def _segment_sum_sc(features, tgt1d, src1d):
    @functools.partial(
        pl.kernel,
        out_type=jax.ShapeDtypeStruct((NC, N_PAD, D), jnp.float32),
        mesh=_sc_mesh,
        scratch_types=[
            pltpu.VMEM((EC,), jnp.int32),
            pltpu.VMEM((EC,), jnp.int32),
            pltpu.VMEM((EC,), jnp.int32),
            pltpu.VMEM((EC,), jnp.int32),
            pltpu.VMEM_SHARED((N_PAD, D), jnp.float32),
            pltpu.VMEM((EC, D), jnp.float32),
            pltpu.VMEM((EC, D), jnp.float32),
            pltpu.SemaphoreType.DMA,
            pltpu.SemaphoreType.DMA,
        ],
    )
    def k(feat_hbm, tgt_hbm, src_hbm, out_hbm, tgt0, tgt1, src0, src1,
          acc_sh, rows0, rows1, sem0, sem1):
        c = lax.axis_index("c")
        s = lax.axis_index("s")
        w = c * NS + s

        def zrow(i, carry):
            for j in range(D // 16):
                rows0[i, pl.ds(j * 16, 16)] = jnp.zeros((16,), jnp.float32)
            return carry

        lax.fori_loop(0, EC, zrow, 0)
        r0 = s * ROWS_PER_TILE
        for j in range(ROWS_PER_TILE // EC):
            pltpu.sync_copy(rows0, acc_sh.at[pl.ds(r0 + j * EC, EC)])
        plsc.subcore_barrier()

        ebase = w * CHUNKS * EC

        def g0wait():
            pltpu.make_async_copy(feat_hbm.at[tgt0], rows0, sem0).wait()

        def g1wait():
            pltpu.make_async_copy(feat_hbm.at[tgt1], rows1, sem1).wait()

        pltpu.sync_copy(tgt_hbm.at[pl.ds(ebase, EC)], tgt0)
        pltpu.sync_copy(src_hbm.at[pl.ds(ebase, EC)], src0)
        pltpu.async_copy(feat_hbm.at[tgt0], rows0, sem0)
        pltpu.sync_copy(tgt_hbm.at[pl.ds(ebase + EC, EC)], tgt1)
        pltpu.sync_copy(src_hbm.at[pl.ds(ebase + EC, EC)], src1)
        pltpu.async_copy(feat_hbm.at[tgt1], rows1, sem1)

        def body2(jj, carry):
            j0 = jj * 2
            g0wait()
            pltpu.sync_copy(rows0, acc_sh.at[src0], add=True)
            pltpu.sync_copy(tgt_hbm.at[pl.ds(ebase + (j0 + 2) * EC, EC)], tgt0)
            pltpu.sync_copy(src_hbm.at[pl.ds(ebase + (j0 + 2) * EC, EC)], src0)
            pltpu.async_copy(feat_hbm.at[tgt0], rows0, sem0)
            g1wait()
            pltpu.sync_copy(rows1, acc_sh.at[src1], add=True)
            pltpu.sync_copy(tgt_hbm.at[pl.ds(ebase + (j0 + 3) * EC, EC)], tgt1)
            pltpu.sync_copy(src_hbm.at[pl.ds(ebase + (j0 + 3) * EC, EC)], src1)
            pltpu.async_copy(feat_hbm.at[tgt1], rows1, sem1)
            return carry

        lax.fori_loop(0, CHUNKS // 2 - 1, body2, 0)
        g0wait()
        pltpu.sync_copy(rows0, acc_sh.at[src0], add=True)
        g1wait()
        pltpu.sync_copy(rows1, acc_sh.at[src1], add=True)
        plsc.subcore_barrier()

        pltpu.sync_copy(acc_sh.at[pl.ds(r0, ROWS_PER_TILE)],
                        out_hbm.at[c, pl.ds(r0, ROWS_PER_TILE)])

    return k(features, tgt1d, src1d)


def _dense_tc(features, partials, W1, W2, b, gamma, beta):
    R = 1000

    def body(f_ref, p_ref, w1_ref, w2_ref, b_ref, g_ref, bt_ref, o_ref):
        x = f_ref[...]
        a = p_ref[0] + p_ref[1]
        dn = (((1,), (1,)), ((), ()))
        y = lax.dot_general(x, w1_ref[...], dn,
                            preferred_element_type=jnp.float32)
        y = y + lax.dot_general(a, w2_ref[...], dn,
                                preferred_element_type=jnp.float32)
        y = y + b_ref[...]
        y = jnp.maximum(y, 0.0)
        y = y * (g_ref[...] * INV_BN) + bt_ref[...]
        n = jnp.sqrt(jnp.sum(y * y, axis=1, keepdims=True))
        o_ref[...] = y / (n + 1e-6)

    return pl.pallas_call(
        body,
        grid=(N_NODES // R,),
        in_specs=[
            pl.BlockSpec((R, D), lambda i: (i, 0)),
            pl.BlockSpec((NC, R, D), lambda i: (0, i, 0)),
            pl.BlockSpec((D, D), lambda i: (0, 0)),
            pl.BlockSpec((D, D), lambda i: (0, 0)),
            pl.BlockSpec((1, D), lambda i: (0, 0)),
            pl.BlockSpec((1, D), lambda i: (0, 0)),
            pl.BlockSpec((1, D), lambda i: (0, 0)),
        ],
        out_specs=pl.BlockSpec((R, D), lambda i: (i, 0)),
        out_shape=jax.ShapeDtypeStruct((N_NODES, D), jnp.float32),
    )(features, partials, W1, W2, b, gamma, beta)


def _gather_sc(u, idx2d):
    @functools.partial(
        pl.kernel,
        out_type=jax.ShapeDtypeStruct((BPAD, D), jnp.float32),
        mesh=_sc_mesh,
        scratch_types=[
            pltpu.VMEM((GCHUNKS, GC), jnp.int32),
            pltpu.VMEM((GC, D), jnp.float32),
            pltpu.VMEM((GC, D), jnp.float32),
            pltpu.SemaphoreType.DMA,
            pltpu.SemaphoreType.DMA,
        ],
    )
    def k(u_hbm, idx_hbm, out_hbm, idx_v, rows0, rows1, sem0, sem1):
        c = lax.axis_index("c")
        s = lax.axis_index("s")
        w = s * NC + c
        base = w * GCHUNKS * GC
        pltpu.sync_copy(idx_hbm.at[pl.ds(w * GCHUNKS, GCHUNKS)], idx_v)
        pltpu.async_copy(u_hbm.at[idx_v.at[0]], rows0, sem0)
        pltpu.async_copy(u_hbm.at[idx_v.at[1]], rows1, sem1)
        for j in range(GCHUNKS):
            rows = rows0 if j % 2 == 0 else rows1
            sem = sem0 if j % 2 == 0 else sem1
            pltpu.make_async_copy(u_hbm.at[idx_v.at[0]], rows, sem).wait()
            pltpu.sync_copy(rows, out_hbm.at[pl.ds(base + j * GC, GC)])
            if j + 2 < GCHUNKS:
                pltpu.async_copy(u_hbm.at[idx_v.at[j + 2]], rows, sem)

    return k(u, idx2d)


def kernel(features, batch, edge_index, W, b, gamma, beta):
    src = edge_index[0].astype(jnp.int32)
    tgt = edge_index[1].astype(jnp.int32)
    bidx = batch.astype(jnp.int32)

    epad = E_PAD - N_EDGES
    pad_src = N_NODES + (jnp.arange(epad, dtype=jnp.int32) % (N_PAD - N_NODES))
    pad_tgt = jnp.arange(epad, dtype=jnp.int32) % N_NODES
    src1d = jnp.concatenate([src, pad_src])
    tgt1d = jnp.concatenate([tgt, pad_tgt])

    partials = _segment_sum_sc(features, tgt1d, src1d)
    W1 = W[:, :D]
    W2 = W[:, D:]
    u = _dense_tc(features, partials, W1, W2, b.reshape(1, D),
                  gamma.reshape(1, D), beta.reshape(1, D))
    bpad = jnp.concatenate(
        [bidx, jnp.zeros((BPAD - N_NODES,), jnp.int32)]).reshape(-1, GC)
    outp = _gather_sc(u, bpad)
    return outp[:N_NODES]


# speedup vs baseline: 2.2961x; 1.0760x over previous
import functools
import math

import jax
import jax.numpy as jnp
from jax import lax
from jax.experimental import pallas as pl
from jax.experimental.pallas import tpu as pltpu
from jax.experimental.pallas import tpu_sc as plsc

N_NODES = 10000
D = 128
N_EDGES = 320000
INV_BN = 1.0 / math.sqrt(1.0 + 1e-5)

NC = 2
NS = 16
NW = NC * NS

EC = 128
E_PAD = 327680
CHUNKS = E_PAD // (NW * EC)
HALF = CHUNKS // 2
PAD_ROW = 10016
N_PAD = 10240
ROWS_PER_TILE = N_PAD // NS

BPAD = 10240
GC = 80
GCHUNKS = BPAD // (NW * GC)

_sc_mesh = plsc.VectorSubcoreMesh(core_axis_name="c", subcore_axis_name="s")


def _segment_sum_sc(features, tgt1d, src1d):
    @functools.partial(
        pl.kernel,
        out_type=jax.ShapeDtypeStruct((NC, N_PAD, D), jnp.float32),
        mesh=_sc_mesh,
        scratch_types=[
            pltpu.VMEM((EC,), jnp.int32),
            pltpu.VMEM((EC,), jnp.int32),
            pltpu.VMEM((EC,), jnp.int32),
            pltpu.VMEM((EC,), jnp.int32),
            pltpu.VMEM_SHARED((N_PAD, D), jnp.float32),
            pltpu.VMEM((EC, D), jnp.float32),
            pltpu.VMEM((EC, D), jnp.float32),
            pltpu.SemaphoreType.DMA,
            pltpu.SemaphoreType.DMA,
        ],
    )
    def k(feat_hbm, tgt_hbm, src_hbm, out_hbm, tgt0, tgt1, src0, src1,
          acc_sh, rows0, rows1, sem0, sem1):
        c = lax.axis_index("c")
        s = lax.axis_index("s")
        w = c * NS + s

        def zrow(i, carry):
            for j in range(D // 16):
                rows0[i, pl.ds(j * 16, 16)] = jnp.zeros((16,), jnp.float32)
            return carry

        lax.fori_loop(0, EC, zrow, 0)
        r0 = s * ROWS_PER_TILE
        for j in range(ROWS_PER_TILE // EC):
            pltpu.sync_copy(rows0, acc_sh.at[pl.ds(r0 + j * EC, EC)])
        plsc.subcore_barrier()

        ebase = w * CHUNKS * EC

        def g0wait():
            pltpu.make_async_copy(feat_hbm.at[tgt0], rows0, sem0).wait()

        def g1wait():
            pltpu.make_async_copy(feat_hbm.at[tgt1], rows1, sem1).wait()

        pltpu.sync_copy(tgt_hbm.at[pl.ds(ebase, EC)], tgt0)
        pltpu.sync_copy(src_hbm.at[pl.ds(ebase, EC)], src0)
        pltpu.async_copy(feat_hbm.at[tgt0], rows0, sem0)
        pltpu.sync_copy(tgt_hbm.at[pl.ds(ebase + EC, EC)], tgt1)
        pltpu.sync_copy(src_hbm.at[pl.ds(ebase + EC, EC)], src1)
        pltpu.async_copy(feat_hbm.at[tgt1], rows1, sem1)

        def body2(jj, carry):
            j0 = jj * 2
            g0wait()
            pltpu.sync_copy(rows0, acc_sh.at[src0], add=True)
            pltpu.sync_copy(tgt_hbm.at[pl.ds(ebase + (j0 + 2) * EC, EC)], tgt0)
            pltpu.sync_copy(src_hbm.at[pl.ds(ebase + (j0 + 2) * EC, EC)], src0)
            pltpu.async_copy(feat_hbm.at[tgt0], rows0, sem0)
            g1wait()
            pltpu.sync_copy(rows1, acc_sh.at[src1], add=True)
            pltpu.sync_copy(tgt_hbm.at[pl.ds(ebase + (j0 + 3) * EC, EC)], tgt1)
            pltpu.sync_copy(src_hbm.at[pl.ds(ebase + (j0 + 3) * EC, EC)], src1)
            pltpu.async_copy(feat_hbm.at[tgt1], rows1, sem1)
            return carry

        lax.fori_loop(0, CHUNKS // 2 - 1, body2, 0)
        g0wait()
        pltpu.sync_copy(rows0, acc_sh.at[src0], add=True)
        g1wait()
        pltpu.sync_copy(rows1, acc_sh.at[src1], add=True)
        plsc.subcore_barrier()

        pltpu.sync_copy(acc_sh.at[pl.ds(r0, ROWS_PER_TILE)],
                        out_hbm.at[c, pl.ds(r0, ROWS_PER_TILE)])

    return k(features, tgt1d, src1d)


def _dense_tc(features, partials, W1, W2, b, gamma, beta):
    R = 1000

    def body(f_ref, p_ref, w1_ref, w2_ref, b_ref, g_ref, bt_ref, o_ref):
        x = f_ref[...]
        a = p_ref[0] + p_ref[1]
        dn = (((1,), (1,)), ((), ()))
        y = lax.dot_general(x, w1_ref[...], dn,
                            preferred_element_type=jnp.float32)
        y = y + lax.dot_general(a, w2_ref[...], dn,
                                preferred_element_type=jnp.float32)
        y = y + b_ref[...]
        y = jnp.maximum(y, 0.0)
        y = y * (g_ref[...] * INV_BN) + bt_ref[...]
        n = jnp.sqrt(jnp.sum(y * y, axis=1, keepdims=True))
        o_ref[...] = y / (n + 1e-6)

    return pl.pallas_call(
        body,
        grid=(N_NODES // R,),
        in_specs=[
            pl.BlockSpec((R, D), lambda i: (i, 0)),
            pl.BlockSpec((NC, R, D), lambda i: (0, i, 0)),
            pl.BlockSpec((D, D), lambda i: (0, 0)),
            pl.BlockSpec((D, D), lambda i: (0, 0)),
            pl.BlockSpec((1, D), lambda i: (0, 0)),
            pl.BlockSpec((1, D), lambda i: (0, 0)),
            pl.BlockSpec((1, D), lambda i: (0, 0)),
        ],
        out_specs=pl.BlockSpec((R, D), lambda i: (i, 0)),
        out_shape=jax.ShapeDtypeStruct((N_NODES, D), jnp.float32),
    )(features, partials, W1, W2, b, gamma, beta)


def _gather_sc(u, idx1d):
    @functools.partial(
        pl.kernel,
        out_type=jax.ShapeDtypeStruct((N_NODES, D), jnp.float32),
        mesh=_sc_mesh,
        scratch_types=[
            pltpu.VMEM((GC,), jnp.int32),
            pltpu.VMEM((GC,), jnp.int32),
            pltpu.VMEM((GC, D), jnp.float32),
            pltpu.VMEM((GC, D), jnp.float32),
            pltpu.SemaphoreType.DMA,
            pltpu.SemaphoreType.DMA,
        ],
    )
    def k(u_hbm, idx_hbm, out_hbm, idx0, idx1, rows0, rows1, sem0, sem1):
        c = lax.axis_index("c")
        s = lax.axis_index("s")
        w = s * NC + c
        base = w * GCHUNKS * GC

        pltpu.sync_copy(idx_hbm.at[pl.ds(base, GC)], idx0)
        pltpu.async_copy(u_hbm.at[idx0], rows0, sem0)

        @pl.when(w < NW - 1)
        def _():
            pltpu.sync_copy(idx_hbm.at[pl.ds(base + GC, GC)], idx1)
            pltpu.async_copy(u_hbm.at[idx1], rows1, sem1)

        pltpu.make_async_copy(u_hbm.at[idx0], rows0, sem0).wait()
        pltpu.sync_copy(rows0, out_hbm.at[pl.ds(base, GC)])

        @pl.when(w < NW - 1)
        def _():
            pltpu.sync_copy(idx_hbm.at[pl.ds(base + 2 * GC, GC)], idx0)
            pltpu.async_copy(u_hbm.at[idx0], rows0, sem0)
            pltpu.make_async_copy(u_hbm.at[idx1], rows1, sem1).wait()
            pltpu.sync_copy(rows1, out_hbm.at[pl.ds(base + GC, GC)])
            pltpu.sync_copy(idx_hbm.at[pl.ds(base + 3 * GC, GC)], idx1)
            pltpu.async_copy(u_hbm.at[idx1], rows1, sem1)
            pltpu.make_async_copy(u_hbm.at[idx0], rows0, sem0).wait()
            pltpu.sync_copy(rows0, out_hbm.at[pl.ds(base + 2 * GC, GC)])
            pltpu.make_async_copy(u_hbm.at[idx1], rows1, sem1).wait()
            pltpu.sync_copy(rows1, out_hbm.at[pl.ds(base + 3 * GC, GC)])

    return k(u, idx1d)


def kernel(features, batch, edge_index, W, b, gamma, beta):
    src = edge_index[0].astype(jnp.int32)
    tgt = edge_index[1].astype(jnp.int32)
    bidx = batch.astype(jnp.int32)

    epad = E_PAD - N_EDGES
    pad_src = N_NODES + (jnp.arange(epad, dtype=jnp.int32) % (N_PAD - N_NODES))
    pad_tgt = jnp.arange(epad, dtype=jnp.int32) % N_NODES
    src1d = jnp.concatenate([src, pad_src])
    tgt1d = jnp.concatenate([tgt, pad_tgt])

    partials = _segment_sum_sc(features, tgt1d, src1d)
    W1 = W[:, :D]
    W2 = W[:, D:]
    u = _dense_tc(features, partials, W1, W2, b.reshape(1, D),
                  gamma.reshape(1, D), beta.reshape(1, D))
    return _gather_sc(u, bidx)


# speedup vs baseline: 2.8496x; 1.2411x over previous
import functools
import math

import jax
import jax.numpy as jnp
from jax import lax
from jax.experimental import pallas as pl
from jax.experimental.pallas import tpu as pltpu
from jax.experimental.pallas import tpu_sc as plsc

N_NODES = 10000
D = 128
N_EDGES = 320000
INV_BN = 1.0 / math.sqrt(1.0 + 1e-5)

NC = 2
NS = 16
NW = NC * NS

EC = 128
E_PAD = 327680
CHUNKS = E_PAD // (NW * EC)
HALF = CHUNKS // 2
PAD_ROW = 10016
N_PAD = 10240
ROWS_PER_TILE = N_PAD // NS

BPAD = 10240
GC = 80
GCHUNKS = BPAD // (NW * GC)

_sc_mesh = plsc.VectorSubcoreMesh(core_axis_name="c", subcore_axis_name="s")


def _segment_sum_sc(features, tgt2d, src2d):
    @functools.partial(
        pl.kernel,
        out_type=jax.ShapeDtypeStruct((NC, N_PAD, D), jnp.float32),
        mesh=_sc_mesh,
        scratch_types=[
            pltpu.VMEM((HALF, EC), jnp.int32),
            pltpu.VMEM((HALF, EC), jnp.int32),
            pltpu.VMEM_SHARED((N_PAD, D), jnp.float32),
            pltpu.VMEM((EC, D), jnp.float32),
            pltpu.VMEM((EC, D), jnp.float32),
            pltpu.SemaphoreType.DMA,
            pltpu.SemaphoreType.DMA,
        ],
    )
    def k(feat_hbm, tgt_hbm, src_hbm, out_hbm, tgt_v, src_v,
          acc_sh, rows0, rows1, sem0, sem1):
        c = lax.axis_index("c")
        s = lax.axis_index("s")
        w = c * NS + s

        def zrow(i, carry):
            for j in range(D // 16):
                rows0[i, pl.ds(j * 16, 16)] = jnp.zeros((16,), jnp.float32)
            return carry

        lax.fori_loop(0, EC, zrow, 0)
        r0 = s * ROWS_PER_TILE
        for j in range(ROWS_PER_TILE // EC):
            pltpu.sync_copy(rows0, acc_sh.at[pl.ds(r0 + j * EC, EC)])
        plsc.subcore_barrier()

        cbase = w * CHUNKS
        for h in range(CHUNKS // HALF):
            hbase = cbase + h * HALF
            pltpu.sync_copy(tgt_hbm.at[pl.ds(hbase, HALF)], tgt_v)
            pltpu.sync_copy(src_hbm.at[pl.ds(hbase, HALF)], src_v)
            pltpu.async_copy(feat_hbm.at[tgt_v.at[0]], rows0, sem0)
            pltpu.async_copy(feat_hbm.at[tgt_v.at[1]], rows1, sem1)

            def body2(jj, carry):
                j0 = jj * 2
                pltpu.make_async_copy(feat_hbm.at[tgt_v.at[0]], rows0,
                                      sem0).wait()
                pltpu.sync_copy(rows0, acc_sh.at[src_v.at[j0]], add=True)
                pltpu.async_copy(feat_hbm.at[tgt_v.at[j0 + 2]], rows0, sem0)
                pltpu.make_async_copy(feat_hbm.at[tgt_v.at[1]], rows1,
                                      sem1).wait()
                pltpu.sync_copy(rows1, acc_sh.at[src_v.at[j0 + 1]], add=True)
                pltpu.async_copy(feat_hbm.at[tgt_v.at[j0 + 3]], rows1, sem1)
                return carry

            lax.fori_loop(0, HALF // 2 - 1, body2, 0)
            pltpu.make_async_copy(feat_hbm.at[tgt_v.at[0]], rows0, sem0).wait()
            pltpu.sync_copy(rows0, acc_sh.at[src_v.at[HALF - 2]], add=True)
            pltpu.make_async_copy(feat_hbm.at[tgt_v.at[1]], rows1, sem1).wait()
            pltpu.sync_copy(rows1, acc_sh.at[src_v.at[HALF - 1]], add=True)
        plsc.subcore_barrier()

        pltpu.sync_copy(acc_sh.at[pl.ds(r0, ROWS_PER_TILE)],
                        out_hbm.at[c, pl.ds(r0, ROWS_PER_TILE)])

    return k(features, tgt2d, src2d)


def _dense_tc(features, partials, W1, W2, b, gamma, beta):
    R = 1000

    def body(f_ref, p_ref, w1_ref, w2_ref, b_ref, g_ref, bt_ref, o_ref):
        x = f_ref[...]
        a = p_ref[0] + p_ref[1]
        dn = (((1,), (1,)), ((), ()))
        y = lax.dot_general(x, w1_ref[...], dn,
                            preferred_element_type=jnp.float32)
        y = y + lax.dot_general(a, w2_ref[...], dn,
                                preferred_element_type=jnp.float32)
        y = y + b_ref[...]
        y = jnp.maximum(y, 0.0)
        y = y * (g_ref[...] * INV_BN) + bt_ref[...]
        n = jnp.sqrt(jnp.sum(y * y, axis=1, keepdims=True))
        o_ref[...] = y / (n + 1e-6)

    return pl.pallas_call(
        body,
        grid=(N_NODES // R,),
        in_specs=[
            pl.BlockSpec((R, D), lambda i: (i, 0)),
            pl.BlockSpec((NC, R, D), lambda i: (0, i, 0)),
            pl.BlockSpec((D, D), lambda i: (0, 0)),
            pl.BlockSpec((D, D), lambda i: (0, 0)),
            pl.BlockSpec((1, D), lambda i: (0, 0)),
            pl.BlockSpec((1, D), lambda i: (0, 0)),
            pl.BlockSpec((1, D), lambda i: (0, 0)),
        ],
        out_specs=pl.BlockSpec((R, D), lambda i: (i, 0)),
        out_shape=jax.ShapeDtypeStruct((N_NODES, D), jnp.float32),
    )(features, partials, W1, W2, b, gamma, beta)


def _gather_sc(u, idx1d):
    @functools.partial(
        pl.kernel,
        out_type=jax.ShapeDtypeStruct((N_NODES, D), jnp.float32),
        mesh=_sc_mesh,
        scratch_types=[
            pltpu.VMEM((GC,), jnp.int32),
            pltpu.VMEM((GC,), jnp.int32),
            pltpu.VMEM((GC, D), jnp.float32),
            pltpu.VMEM((GC, D), jnp.float32),
            pltpu.SemaphoreType.DMA,
            pltpu.SemaphoreType.DMA,
        ],
    )
    def k(u_hbm, idx_hbm, out_hbm, idx0, idx1, rows0, rows1, sem0, sem1):
        c = lax.axis_index("c")
        s = lax.axis_index("s")
        w = s * NC + c
        base = w * GCHUNKS * GC

        pltpu.sync_copy(idx_hbm.at[pl.ds(base, GC)], idx0)
        pltpu.async_copy(u_hbm.at[idx0], rows0, sem0)

        @pl.when(w < NW - 1)
        def _():
            pltpu.sync_copy(idx_hbm.at[pl.ds(base + GC, GC)], idx1)
            pltpu.async_copy(u_hbm.at[idx1], rows1, sem1)

        pltpu.make_async_copy(u_hbm.at[idx0], rows0, sem0).wait()
        pltpu.sync_copy(rows0, out_hbm.at[pl.ds(base, GC)])

        @pl.when(w < NW - 1)
        def _():
            pltpu.sync_copy(idx_hbm.at[pl.ds(base + 2 * GC, GC)], idx0)
            pltpu.async_copy(u_hbm.at[idx0], rows0, sem0)
            pltpu.make_async_copy(u_hbm.at[idx1], rows1, sem1).wait()
            pltpu.sync_copy(rows1, out_hbm.at[pl.ds(base + GC, GC)])
            pltpu.sync_copy(idx_hbm.at[pl.ds(base + 3 * GC, GC)], idx1)
            pltpu.async_copy(u_hbm.at[idx1], rows1, sem1)
            pltpu.make_async_copy(u_hbm.at[idx0], rows0, sem0).wait()
            pltpu.sync_copy(rows0, out_hbm.at[pl.ds(base + 2 * GC, GC)])
            pltpu.make_async_copy(u_hbm.at[idx1], rows1, sem1).wait()
            pltpu.sync_copy(rows1, out_hbm.at[pl.ds(base + 3 * GC, GC)])

    return k(u, idx1d)


def kernel(features, batch, edge_index, W, b, gamma, beta):
    src = edge_index[0].astype(jnp.int32)
    tgt = edge_index[1].astype(jnp.int32)
    bidx = batch.astype(jnp.int32)

    epad = E_PAD - N_EDGES
    pad_src = N_NODES + (jnp.arange(epad, dtype=jnp.int32) % (N_PAD - N_NODES))
    pad_tgt = jnp.arange(epad, dtype=jnp.int32) % N_NODES
    src2d = jnp.concatenate([src, pad_src]).reshape(-1, EC)
    tgt2d = jnp.concatenate([tgt, pad_tgt]).reshape(-1, EC)

    partials = _segment_sum_sc(features, tgt2d, src2d)
    W1 = W[:, :D]
    W2 = W[:, D:]
    u = _dense_tc(features, partials, W1, W2, b.reshape(1, D),
                  gamma.reshape(1, D), beta.reshape(1, D))
    return _gather_sc(u, bidx)
